# baseline (speedup 1.0000x reference)
import functools

import jax
import jax.numpy as jnp
from jax import lax
from jax.experimental import pallas as pl
from jax.experimental.pallas import tpu as pltpu
from jax.experimental.pallas import tpu_sc as plsc

B, C, H, W = 32, 192, 56, 56
G = C // 2
ROWS = B * H
NW = 32
RPW = ROWS // NW
NL = 16
NV = C // NL


def kernel(x1, x2):
    xt1 = jnp.transpose(x1, (0, 2, 3, 1)).reshape(ROWS, W, C)
    xt2 = jnp.transpose(x2, (0, 2, 3, 1)).reshape(ROWS, W, C)

    mesh = plsc.VectorSubcoreMesh(core_axis_name="c", subcore_axis_name="s")

    @functools.partial(
        pl.kernel,
        out_type=[
            jax.ShapeDtypeStruct((ROWS, W, C), jnp.float32),
            jax.ShapeDtypeStruct((ROWS, W, C), jnp.float32),
        ],
        mesh=mesh,
        scratch_types=[
            pltpu.VMEM((2, 2, W, C), jnp.float32),
            pltpu.VMEM((2, 2, W, C), jnp.float32),
            pltpu.SemaphoreType.DMA,
            pltpu.SemaphoreType.DMA,
            pltpu.SemaphoreType.DMA,
            pltpu.SemaphoreType.DMA,
        ],
        compiler_params=pltpu.CompilerParams(
            use_tc_tiling_on_sc=True, needs_layout_passes=False),
    )
    def shuffle(x1_hbm, x2_hbm, o1_hbm, o2_hbm, ibuf, obuf,
                in0, in1, out0, out1):
        wid = lax.axis_index("s") * 2 + lax.axis_index("c")
        base = wid * RPW
        in_sems = (in0, in1)
        out_sems = (out0, out1)

        lane = lax.iota(jnp.int32, NL)
        two = jnp.full((NL,), 2, jnp.int32)
        alt = lax.rem(lane, two)
        flr = lax.div(lane, two)
        cidx = [[lax.add(flr, jnp.full((NL,), G * o + (NL // 2) * v,
                                       jnp.int32))
                 for v in range(NV)]
                for o in range(2)]

        def fire_in(t, p):
            r = base + t
            pltpu.async_copy(x1_hbm.at[r], ibuf.at[p, 0], in_sems[p])
            pltpu.async_copy(x2_hbm.at[r], ibuf.at[p, 1], in_sems[p])

        def wait_in(t, p):
            r = base + t
            pltpu.make_async_copy(
                x1_hbm.at[r], ibuf.at[p, 0], in_sems[p]).wait()
            pltpu.make_async_copy(
                x2_hbm.at[r], ibuf.at[p, 1], in_sems[p]).wait()

        def fire_out(t, p, o):
            r = base + t
            dst = o1_hbm if o == 0 else o2_hbm
            pltpu.async_copy(obuf.at[p, o], dst.at[r], out_sems[p])

        def wait_out(t, p):
            r = base + t
            pltpu.make_async_copy(
                obuf.at[p, 0], o1_hbm.at[r], out_sems[p]).wait()
            pltpu.make_async_copy(
                obuf.at[p, 1], o2_hbm.at[r], out_sems[p]).wait()

        def compute(p, o):
            src = ibuf.at[p]

            @plsc.parallel_loop(0, W, 1, unroll=4)
            def wbody(w):
                wv = jnp.full((NL,), w, jnp.int32)
                vals = [plsc.load_gather(src, [alt, wv, cidx[o][v]])
                        for v in range(NV)]
                for v in range(NV):
                    obuf[p, o, w, pl.ds(NL * v, NL)] = vals[v]

        fire_in(0, 0)
        fire_in(1, 1)

        def body(t2, carry):
            for p in (0, 1):
                t = 2 * t2 + p
                wait_in(t, p)

                @pl.when(t2 > 0)
                def _():
                    wait_out(t - 2, p)

                compute(p, 0)
                fire_out(t, p, 0)
                compute(p, 1)
                fire_out(t, p, 1)

                @pl.when(t2 < RPW // 2 - 1)
                def _():
                    fire_in(t + 2, p)
            return carry

        lax.fori_loop(0, RPW // 2, body, 0)
        wait_out(RPW - 2, 0)
        wait_out(RPW - 1, 1)

    o1, o2 = shuffle(xt1, xt2)
    o1 = jnp.transpose(o1.reshape(B, H, W, C), (0, 3, 1, 2))
    o2 = jnp.transpose(o2.reshape(B, H, W, C), (0, 3, 1, 2))
    return o1, o2
